# baseline (speedup 1.0000x reference)
import functools

import jax
import jax.numpy as jnp
from jax.experimental import pallas as pl
from jax.experimental.pallas import tpu as pltpu

_LEAKY = 0.5


def _pass1_body(adj_ref, iemb_ref, uemb_ref, acc_ref, adj8_ref, scr_ref,
                *, nsteps, d):
    r = pl.program_id(0)

    @pl.when(r == 0)
    def _init():
        scr_ref[...] = jnp.zeros_like(scr_ref)

    adj = adj_ref[...]
    adjb = adj.astype(jnp.bfloat16)
    adj8_ref[...] = adj.astype(jnp.int8)

    ti = jnp.dot(adjb, iemb_ref[...].astype(jnp.bfloat16),
                 preferred_element_type=jnp.float32)
    x = jnp.concatenate([uemb_ref[...], ti], axis=1).astype(jnp.bfloat16)
    scr_ref[...] += jax.lax.dot_general(
        x, adjb, (((0,), (0,)), ((), ())), preferred_element_type=jnp.float32)

    @pl.when(r == nsteps - 1)
    def _flush():
        acc = scr_ref[...]
        hi = acc[d:, :]
        acc_ref[:d, :] = acc[:d, :]
        acc_ref[d:, :] = jnp.where(hi >= 0, hi, _LEAKY * hi)


def _pass2_body(adj8_ref, tuT_ref, uemb_ref, uall_ref, *, d):
    hu = jax.lax.dot_general(
        adj8_ref[...].astype(jnp.bfloat16), tuT_ref[...].astype(jnp.bfloat16),
        (((1,), (1,)), ((), ())), preferred_element_type=jnp.float32)
    hu = jnp.where(hu >= 0, hu, _LEAKY * hu)
    uall_ref[:, :d] = uemb_ref[...]
    uall_ref[:, d:2 * d] = hu
    uall_ref[:, 2 * d:] = hu


@functools.partial(jax.jit, static_argnames=("stripe", "stripe2"))
def _dhcf(adj, user_emb, item_emb, stripe=128, stripe2=1024):
    n_u, n_i = adj.shape
    d = user_emb.shape[1]
    nsteps = n_u // stripe

    params = pltpu.CompilerParams(dimension_semantics=("arbitrary",))

    acc, adj8 = pl.pallas_call(
        functools.partial(_pass1_body, nsteps=nsteps, d=d),
        grid=(nsteps,),
        in_specs=[
            pl.BlockSpec((stripe, n_i), lambda r: (r, 0)),
            pl.BlockSpec((n_i, d), lambda r: (0, 0)),
            pl.BlockSpec((stripe, d), lambda r: (r, 0)),
        ],
        out_specs=[
            pl.BlockSpec((2 * d, n_i), lambda r: (0, 0)),
            pl.BlockSpec((stripe, n_i), lambda r: (r, 0)),
        ],
        out_shape=[
            jax.ShapeDtypeStruct((2 * d, n_i), jnp.float32),
            jax.ShapeDtypeStruct((n_u, n_i), jnp.int8),
        ],
        scratch_shapes=[pltpu.VMEM((2 * d, n_i), jnp.float32)],
        compiler_params=params,
    )(adj, item_emb, user_emb)

    user_all = pl.pallas_call(
        functools.partial(_pass2_body, d=d),
        grid=(n_u // stripe2,),
        in_specs=[
            pl.BlockSpec((stripe2, n_i), lambda r: (r, 0)),
            pl.BlockSpec((d, n_i), lambda r: (0, 0)),
            pl.BlockSpec((stripe2, d), lambda r: (r, 0)),
        ],
        out_specs=pl.BlockSpec((stripe2, 3 * d), lambda r: (r, 0)),
        out_shape=jax.ShapeDtypeStruct((n_u, 3 * d), jnp.float32),
        compiler_params=params,
    )(adj8, acc, user_emb)

    h_i = acc[d:].T
    item_all = jnp.concatenate([item_emb, h_i, h_i], axis=1)
    return user_all, item_all


def kernel(adj, user_emb, item_emb):
    return _dhcf(adj, user_emb, item_emb)


# speedup vs baseline: 1.0266x; 1.0266x over previous
import functools

import jax
import jax.numpy as jnp
from jax.experimental import pallas as pl
from jax.experimental.pallas import tpu as pltpu

_LEAKY = 0.5


def _pass1_body(adj_ref, iemb_ref, uemb_ref, acc_ref, adj8_ref, scr_ref,
                *, nsteps, d):
    r = pl.program_id(0)

    @pl.when(r == 0)
    def _init():
        scr_ref[...] = jnp.zeros_like(scr_ref)

    adj = adj_ref[...]
    adjb = adj.astype(jnp.bfloat16)
    adj8_ref[...] = adj.astype(jnp.int8)

    ti = jnp.dot(adjb, iemb_ref[...],
                 preferred_element_type=jnp.float32)
    x = jnp.concatenate([uemb_ref[...], ti], axis=1).astype(jnp.bfloat16)
    scr_ref[...] += jax.lax.dot_general(
        x, adjb, (((0,), (0,)), ((), ())), preferred_element_type=jnp.float32)

    @pl.when(r == nsteps - 1)
    def _flush():
        acc = scr_ref[...]
        hi = acc[d:, :]
        acc_ref[:d, :] = acc[:d, :].astype(jnp.bfloat16)
        acc_ref[d:, :] = jnp.where(hi >= 0, hi, _LEAKY * hi).astype(jnp.bfloat16)


def _pass2_body(adj8_ref, tuT_ref, uemb_ref, uall_ref, *, d):
    hu = jax.lax.dot_general(
        adj8_ref[...].astype(jnp.bfloat16), tuT_ref[...],
        (((1,), (1,)), ((), ())), preferred_element_type=jnp.float32)
    hu = jnp.where(hu >= 0, hu, _LEAKY * hu)
    uall_ref[:, :d] = uemb_ref[...]
    uall_ref[:, d:2 * d] = hu
    uall_ref[:, 2 * d:] = hu


@functools.partial(jax.jit, static_argnames=("stripe", "stripe2"))
def _dhcf(adj, user_emb, item_emb, stripe=256, stripe2=1024):
    n_u, n_i = adj.shape
    d = user_emb.shape[1]
    nsteps = n_u // stripe

    params = pltpu.CompilerParams(dimension_semantics=("arbitrary",))

    acc, adj8 = pl.pallas_call(
        functools.partial(_pass1_body, nsteps=nsteps, d=d),
        grid=(nsteps,),
        in_specs=[
            pl.BlockSpec((stripe, n_i), lambda r: (r, 0)),
            pl.BlockSpec((n_i, d), lambda r: (0, 0)),
            pl.BlockSpec((stripe, d), lambda r: (r, 0)),
        ],
        out_specs=[
            pl.BlockSpec((2 * d, n_i), lambda r: (0, 0)),
            pl.BlockSpec((stripe, n_i), lambda r: (r, 0)),
        ],
        out_shape=[
            jax.ShapeDtypeStruct((2 * d, n_i), jnp.bfloat16),
            jax.ShapeDtypeStruct((n_u, n_i), jnp.int8),
        ],
        scratch_shapes=[pltpu.VMEM((2 * d, n_i), jnp.float32)],
        compiler_params=params,
    )(adj, item_emb.astype(jnp.bfloat16), user_emb)

    user_all = pl.pallas_call(
        functools.partial(_pass2_body, d=d),
        grid=(n_u // stripe2,),
        in_specs=[
            pl.BlockSpec((stripe2, n_i), lambda r: (r, 0)),
            pl.BlockSpec((d, n_i), lambda r: (0, 0)),
            pl.BlockSpec((stripe2, d), lambda r: (r, 0)),
        ],
        out_specs=pl.BlockSpec((stripe2, 3 * d), lambda r: (r, 0)),
        out_shape=jax.ShapeDtypeStruct((n_u, 3 * d), jnp.float32),
        compiler_params=params,
    )(adj8, acc, user_emb)

    h_i = acc[d:].astype(jnp.float32).T
    item_all = jnp.concatenate([item_emb, h_i, h_i], axis=1)
    return user_all, item_all


def kernel(adj, user_emb, item_emb):
    return _dhcf(adj, user_emb, item_emb)
